# Initial kernel scaffold; baseline (speedup 1.0000x reference)
import jax
import jax.numpy as jnp
from jax.experimental import pallas as pl


def kernel(x, edge_index, W0, as0, ad0, b0, Wr1, br1, W1, as1, ad1, b1, Wr2, br2, W2, as2, ad2, b2, Wl, bl):
    raise NotImplementedError("write your pallas kernel here")


# speedup vs baseline: 4.4491x; 4.4491x over previous
import functools
import math

import jax
import jax.numpy as jnp
from jax import lax
from jax.experimental import pallas as pl
from jax.experimental.pallas import tpu as pltpu
from jax.experimental.pallas import tpu_sc as plsc

_N = 10000
_E_RAW = 160000
_ET = _E_RAW + _N
_HD = 1024
_H = 8
_D = 128
_NPAD = 10240
_CPN = 64
_NSTEP = 5
_NW = 32
_NRANGES = _NSTEP * _NW


def _proj_body(x_ref, w_ref, o_ref):
    o_ref[...] = jnp.dot(x_ref[...], w_ref[...],
                         preferred_element_type=jnp.float32)


def _proj_tc(xp, wcat):
    grid = _NPAD // 128
    return pl.pallas_call(
        _proj_body,
        grid=(grid,),
        in_specs=[
            pl.BlockSpec((128, 128), lambda i: (i, 0)),
            pl.BlockSpec((128, 1152), lambda i: (0, 0)),
        ],
        out_specs=pl.BlockSpec((128, 1152), lambda i: (i, 0)),
        out_shape=jax.ShapeDtypeStruct((_NPAD, 1152), jnp.float32),
    )(xp, wcat)


def _post_body(g_ref, b_ref, w_ref, br_ref, o_ref):
    t = g_ref[...] + b_ref[0:1, :]
    t = 0.5 * t * (1.0 + lax.erf(t * (1.0 / math.sqrt(2.0))))
    o_ref[...] = (jnp.dot(t, w_ref[...], preferred_element_type=jnp.float32)
                  + br_ref[0:1, :])


def _post_tc(gat_raw, bpad, wr, brpad):
    grid = _NPAD // 128
    return pl.pallas_call(
        _post_body,
        grid=(grid,),
        in_specs=[
            pl.BlockSpec((128, _HD), lambda i: (i, 0)),
            pl.BlockSpec((8, _HD), lambda i: (0, 0)),
            pl.BlockSpec((_HD, 128), lambda i: (0, 0)),
            pl.BlockSpec((8, 128), lambda i: (0, 0)),
        ],
        out_specs=pl.BlockSpec((128, 128), lambda i: (i, 0)),
        out_shape=jax.ShapeDtypeStruct((_NPAD, 128), jnp.float32),
    )(gat_raw, bpad, wr, brpad)


def _sc_edge_body(srcs, dsts, eoff, sdt, h, out,
                  eoffv, idxs, idxd, sdsrc, sddst, den, acc, rows, sem):
    wid = lax.axis_index("s") * 2 + lax.axis_index("c")
    pltpu.sync_copy(eoff, eoffv)

    iota_f = lax.iota(jnp.int32, 16).astype(jnp.float32)
    headmask_f = jnp.minimum(jnp.maximum(7.5 - iota_f, 0.0), 1.0)
    zero16 = jnp.zeros((16,), jnp.float32)

    def _ex_for_lane(l, lo, hi, e0):
        al = sdsrc[l, pl.ds(0, 16)] + sddst[l, pl.ds(16, 16)]
        al = jnp.maximum(al, 0.0) + 0.2 * jnp.minimum(al, 0.0)
        ex = jnp.exp(al)
        ee = e0 + l
        vf = jnp.where(ee >= lo, 1.0, 0.0) * jnp.where(ee < hi, 1.0, 0.0)
        return ex * (headmask_f * vf)

    def _lane_dst(dvec_f, l):
        sel = jnp.maximum(1.0 - jnp.abs(iota_f - l.astype(jnp.float32)), 0.0)
        dl = jnp.sum(dvec_f * sel).astype(jnp.int32)
        return lax.max(lax.min(dl, _CPN - 1), 0)

    def _step(step, _carry):
        tr = step * _NW + wid
        base = tr * _CPN
        offv = eoffv[tr, :]
        lo = offv[0]
        hi = offv[1]
        c0 = lo // 16
        c1 = (hi + 15) // 16

        def _zero(i, _):
            den[i, :] = zero16
            for v in range(_HD // 16):
                acc[i, pl.ds(v * 16, 16)] = zero16
            return 0

        lax.fori_loop(0, _CPN, _zero, 0)

        def _chunk_a(c, _):
            e0 = c * 16
            pltpu.sync_copy(srcs.at[pl.ds(e0, 16)], idxs)
            pltpu.sync_copy(dsts.at[pl.ds(e0, 16)], idxd)
            pltpu.async_copy(sdt.at[idxs], sdsrc, sem).wait()
            pltpu.async_copy(sdt.at[idxd], sddst, sem).wait()
            dvec_f = (idxd[...] - base).astype(jnp.float32)

            def _lane_a(l, _):
                ex = _ex_for_lane(l, lo, hi, e0)
                dlc = _lane_dst(dvec_f, l)
                den[dlc, :] = den[dlc, :] + ex
                return 0

            lax.fori_loop(0, 16, _lane_a, 0)
            return 0

        lax.fori_loop(c0, c1, _chunk_a, 0)

        def _chunk_b(c, _):
            e0 = c * 16
            pltpu.sync_copy(srcs.at[pl.ds(e0, 16)], idxs)
            pltpu.sync_copy(dsts.at[pl.ds(e0, 16)], idxd)
            pltpu.async_copy(sdt.at[idxs], sdsrc, sem).wait()
            pltpu.async_copy(sdt.at[idxd], sddst, sem).wait()
            pltpu.async_copy(h.at[idxs], rows, sem).wait()
            dvec_f = (idxd[...] - base).astype(jnp.float32)

            def _lane_b(l, _):
                ex = _ex_for_lane(l, lo, hi, e0)
                dlc = _lane_dst(dvec_f, l)
                w = ex / (den[dlc, :] + 1e-16)
                for k in range(_H):
                    wk = jnp.broadcast_to(w[k], (16,))
                    for v in range(8):
                        off = (k * 8 + v) * 16
                        acc[dlc, pl.ds(off, 16)] = (
                            acc[dlc, pl.ds(off, 16)]
                            + wk * rows[l, pl.ds(off, 16)])
                return 0

            lax.fori_loop(0, 16, _lane_b, 0)
            return 0

        lax.fori_loop(c0, c1, _chunk_b, 0)

        pltpu.sync_copy(acc, out.at[pl.ds(base, _CPN)])
        return 0

    lax.fori_loop(0, _NSTEP, _step, 0)


def _sc_edge(srcs, dsts, eoff, sdt, h):
    mesh = plsc.VectorSubcoreMesh(core_axis_name="c", subcore_axis_name="s")
    kern = functools.partial(
        pl.kernel,
        mesh=mesh,
        out_type=jax.ShapeDtypeStruct((_NPAD, _HD), jnp.float32),
        scratch_types=[
            pltpu.VMEM((_NRANGES, 16), jnp.int32),
            pltpu.VMEM((16,), jnp.int32),
            pltpu.VMEM((16,), jnp.int32),
            pltpu.VMEM((16, 128), jnp.float32),
            pltpu.VMEM((16, 128), jnp.float32),
            pltpu.VMEM((_CPN, 16), jnp.float32),
            pltpu.VMEM((_CPN, _HD), jnp.float32),
            pltpu.VMEM((16, _HD), jnp.float32),
            pltpu.SemaphoreType.DMA,
        ],
        compiler_params=pltpu.CompilerParams(needs_layout_passes=False),
    )(_sc_edge_body)
    return kern(srcs, dsts, eoff, sdt, h)


def _block_diag_scores(a_src, a_dst):
    a = jnp.zeros((_HD, 32), jnp.float32)
    for k in range(_H):
        a = a.at[k * _D:(k + 1) * _D, k].set(a_src[0, k, :])
        a = a.at[k * _D:(k + 1) * _D, 16 + k].set(a_dst[0, k, :])
    return a


def kernel(x, edge_index, W0, as0, ad0, b0, Wr1, br1, W1, as1, ad1, b1,
           Wr2, br2, W2, as2, ad2, b2, Wl, bl):
    f32 = jnp.float32

    loops = jnp.arange(_N, dtype=edge_index.dtype)
    src = jnp.concatenate([edge_index[0], loops])
    dst = jnp.concatenate([edge_index[1], loops])
    order = jnp.argsort(dst)
    srcs = src[order].astype(jnp.int32)
    dsts = dst[order].astype(jnp.int32)
    range_starts = jnp.arange(_NRANGES + 1, dtype=jnp.int32) * _CPN
    eoff1 = jnp.searchsorted(dsts, range_starts).astype(jnp.int32)
    eoff = jnp.pad(jnp.stack([eoff1[:-1], eoff1[1:]], axis=1),
                   ((0, 0), (0, 14)))

    xp = jnp.pad(x.astype(f32), ((0, _NPAD - _N), (0, 0)))

    layers = [
        (W0, as0, ad0, b0, Wr1, br1),
        (W1, as1, ad1, b1, Wr2, br2),
        (W2, as2, ad2, b2, Wl, bl),
    ]
    for (W, a_s, a_d, b, Wr, br) in layers:
        amat = _block_diag_scores(a_s.astype(f32), a_d.astype(f32))
        wsd = W.astype(f32) @ amat
        wcat = jnp.concatenate(
            [W.astype(f32), jnp.pad(wsd, ((0, 0), (0, 96)))], axis=1)
        hw = _proj_tc(xp, wcat)
        h = hw[:, :_HD]
        sdtab = hw[:, _HD:]
        gat_raw = _sc_edge(srcs, dsts, eoff, sdtab, h)
        bpad = jnp.broadcast_to(b.astype(f32)[None, :], (8, _HD))
        brpad = jnp.broadcast_to(br.astype(f32)[None, :], (8, 128))
        xp = _post_tc(gat_raw, bpad, Wr.astype(f32), brpad)

    return xp[:_N]
